# baseline (speedup 1.0000x reference)
import functools

import jax
import jax.numpy as jnp
from jax import lax
from jax.experimental import pallas as pl
from jax.experimental.pallas import tpu as pltpu
from jax.experimental.pallas import tpu_sc as plsc

C = 10000
F = 2048
B = 16384
MOMENTUM = 0.9

NCORE = 2
NSUB_CORES = 16
LANES = 16

FC = 128
NCHUNK = (F // NCORE) // FC
TPB = B // NSUB_CORES
SB = 128
NSB = TPB // SB
QR = 128
NQ = TPB // QR
CPAD = 10240
RPT = CPAD // NSUB_CORES
HRPT = RPT // 2
ZR = 64

BCHK_N = 512
CBLK_FIN = 512


def _norm_body(feat_ref, out_ref):
    f = feat_ref[...]
    ss = jnp.sum(f * f, axis=1, keepdims=True)
    inv = 1.0 / jnp.maximum(jnp.sqrt(ss), 1e-12)
    out_ref[...] = f * inv


_norms = pl.pallas_call(
    _norm_body,
    grid=(B // BCHK_N,),
    in_specs=[pl.BlockSpec((BCHK_N, F), lambda j: (j, 0))],
    out_specs=pl.BlockSpec((BCHK_N, F), lambda j: (j, 0)),
    out_shape=jax.ShapeDtypeStruct((B, F), jnp.float32),
    compiler_params=pltpu.CompilerParams(
        dimension_semantics=("arbitrary",),
    ),
)


def _fin_body(sums_ref, mem_ref, out_ref):
    s = sums_ref[...]
    ssc = jnp.sum(s * s, axis=1, keepdims=True)
    featm = s * (1.0 / jnp.maximum(jnp.sqrt(ssc), 1e-12))
    mem = mem_ref[...]
    new = MOMENTUM * mem + (1.0 - MOMENTUM) * featm
    ssn = jnp.sum(new * new, axis=1, keepdims=True)
    newn = new * (1.0 / jnp.maximum(jnp.sqrt(ssn), 1e-12))
    out_ref[...] = jnp.where(ssc > 0, newn, mem)


_finalize = pl.pallas_call(
    _fin_body,
    grid=((C + CBLK_FIN - 1) // CBLK_FIN,),
    in_specs=[
        pl.BlockSpec((CBLK_FIN, F), lambda i: (i, 0)),
        pl.BlockSpec((CBLK_FIN, F), lambda i: (i, 0)),
    ],
    out_specs=pl.BlockSpec((CBLK_FIN, F), lambda i: (i, 0)),
    out_shape=jax.ShapeDtypeStruct((C, F), jnp.float32),
    compiler_params=pltpu.CompilerParams(
        dimension_semantics=("arbitrary",),
    ),
)


def _sc_body(feat_hbm, ids_hbm, out_hbm,
             fbuf, idsb, zbuf,
             acc, feat_sem, scat_sem, flush_sem):
    c = lax.axis_index("c")
    s = lax.axis_index("s")
    row_base = s * TPB
    col_base = c * (F // NCORE)
    stripe = s * RPT

    pltpu.sync_copy(ids_hbm.at[s], idsb)

    zv = jnp.zeros((LANES,), jnp.float32)

    def zrow(r, carry):
        for jj in range(FC // LANES):
            zbuf[r, pl.ds(jj * LANES, LANES)] = zv
        return carry

    lax.fori_loop(0, ZR, zrow, 0)

    def _feat_dma(buf_idx, gq):
        gqw = lax.rem(gq, NCHUNK * NQ)
        fch = lax.div(gqw, NQ)
        q = lax.rem(gqw, NQ)
        return pltpu.make_async_copy(
            feat_hbm.at[pl.ds(row_base + q * QR, QR),
                        pl.ds(col_base + fch * FC, FC)],
            fbuf.at[buf_idx], feat_sem)

    def _fire_scat(buf_idx, q):
        pltpu.async_copy(
            fbuf.at[buf_idx], acc.at[idsb.at[q]], scat_sem, add=True)

    def _drain_scat():
        pltpu.make_async_copy(
            fbuf.at[0], acc.at[idsb.at[0]], scat_sem).wait()

    for z in range(RPT // ZR):
        pltpu.sync_copy(zbuf, acc.at[pl.ds(stripe + z * ZR, ZR)])
    _feat_dma(0, 0).start()
    plsc.subcore_barrier()

    def chunk_body(fchunk, carry):
        gq0 = fchunk * NQ
        col0 = col_base + fchunk * FC

        _feat_dma(0, gq0).wait()

        @pl.when(fchunk >= 1)
        def _():
            for h in range(2):
                pltpu.make_async_copy(
                    acc.at[pl.ds(stripe + h * HRPT, HRPT)],
                    out_hbm.at[pl.ds(stripe + h * HRPT, HRPT),
                               pl.ds(col0, FC)],
                    flush_sem).wait()
                for z in range(HRPT // ZR):
                    pltpu.sync_copy(
                        zbuf,
                        acc.at[pl.ds(stripe + h * HRPT + z * ZR, ZR)])

        plsc.subcore_barrier()

        _fire_scat(0, 0)
        _feat_dma(1, gq0 + 1).start()
        for q in range(1, NQ):
            b = q % 2
            _feat_dma(b, gq0 + q).wait()
            _fire_scat(b, q)
            _drain_scat()
            _feat_dma(1 - b, gq0 + q + 1).start()
        _drain_scat()

        plsc.subcore_barrier()
        for h in range(2):
            pltpu.async_copy(
                acc.at[pl.ds(stripe + h * HRPT, HRPT)],
                out_hbm.at[pl.ds(stripe + h * HRPT, HRPT),
                           pl.ds(col0, FC)],
                flush_sem)
        return carry

    lax.fori_loop(0, NCHUNK, chunk_body, 0)

    for h in range(2):
        pltpu.make_async_copy(
            acc.at[pl.ds(stripe + h * HRPT, HRPT)],
            out_hbm.at[pl.ds(stripe + h * HRPT, HRPT), pl.ds(col_base, FC)],
            flush_sem).wait()
    _feat_dma(0, 0).wait()


_sc_scatter = functools.partial(
    pl.kernel,
    out_type=jax.ShapeDtypeStruct((CPAD, F), jnp.float32),
    mesh=plsc.VectorSubcoreMesh(core_axis_name="c", subcore_axis_name="s"),
    cost_estimate=pl.CostEstimate(
        flops=0, transcendentals=0, bytes_accessed=400_000_000),
    scratch_types=[
        pltpu.VMEM((2, QR, FC), jnp.float32),
        pltpu.VMEM((NSB, SB), jnp.int32),
        pltpu.VMEM((ZR, FC), jnp.float32),
        pltpu.VMEM_SHARED((CPAD, FC), jnp.float32),
        pltpu.SemaphoreType.DMA,
        pltpu.SemaphoreType.DMA,
        pltpu.SemaphoreType.DMA,
    ],
)(_sc_body)


def kernel(features_v, features_r, vis_memory, ir_memory, ids_v, ids_r):
    fn_v = _norms(features_v)
    sums_v = _sc_scatter(fn_v, ids_v.reshape(NSUB_CORES, NSB, SB))
    fn_r = _norms(features_r)
    sums_r = _sc_scatter(fn_r, ids_r.reshape(NSUB_CORES, NSB, SB))
    vis_new = _finalize(sums_v, vis_memory)
    ir_new = _finalize(sums_r, ir_memory)
    return (vis_new, ir_new)


# speedup vs baseline: 1.0258x; 1.0258x over previous
import functools

import jax
import jax.numpy as jnp
from jax import lax
from jax.experimental import pallas as pl
from jax.experimental.pallas import tpu as pltpu
from jax.experimental.pallas import tpu_sc as plsc

C = 10000
F = 2048
B = 16384
MOMENTUM = 0.9

NCORE = 2
NSUB_CORES = 16
LANES = 16

FC = 128
NCHUNK = (F // NCORE) // FC
TPB = B // NSUB_CORES
SB = 128
NSB = TPB // SB
QR = 128
NQ = TPB // QR
CPAD = 10240
RPT = CPAD // NSUB_CORES
HRPT = RPT // 4
ZR = 80

BCHK_N = 512
CBLK_FIN = 512


def _norm_body(feat_ref, out_ref):
    f = feat_ref[...]
    ss = jnp.sum(f * f, axis=1)
    inv = 1.0 / jnp.maximum(jnp.sqrt(ss), 1e-12)
    out_ref[...] = inv.reshape(1, 1, BCHK_N)


_norms = pl.pallas_call(
    _norm_body,
    grid=(B // BCHK_N,),
    in_specs=[pl.BlockSpec((BCHK_N, F), lambda j: (j, 0))],
    out_specs=pl.BlockSpec((1, 1, BCHK_N), lambda j: (j, 0, 0)),
    out_shape=jax.ShapeDtypeStruct((B // BCHK_N, 1, BCHK_N), jnp.float32),
    compiler_params=pltpu.CompilerParams(
        dimension_semantics=("arbitrary",),
    ),
)


def _fin_body(sums_ref, mem_ref, out_ref):
    s = sums_ref[...]
    ssc = jnp.sum(s * s, axis=1, keepdims=True)
    featm = s * (1.0 / jnp.maximum(jnp.sqrt(ssc), 1e-12))
    mem = mem_ref[...]
    new = MOMENTUM * mem + (1.0 - MOMENTUM) * featm
    ssn = jnp.sum(new * new, axis=1, keepdims=True)
    newn = new * (1.0 / jnp.maximum(jnp.sqrt(ssn), 1e-12))
    out_ref[...] = jnp.where(ssc > 0, newn, mem)


_finalize = pl.pallas_call(
    _fin_body,
    grid=((C + CBLK_FIN - 1) // CBLK_FIN,),
    in_specs=[
        pl.BlockSpec((CBLK_FIN, F), lambda i: (i, 0)),
        pl.BlockSpec((CBLK_FIN, F), lambda i: (i, 0)),
    ],
    out_specs=pl.BlockSpec((CBLK_FIN, F), lambda i: (i, 0)),
    out_shape=jax.ShapeDtypeStruct((C, F), jnp.float32),
    compiler_params=pltpu.CompilerParams(
        dimension_semantics=("arbitrary",),
    ),
)


def _sc_body(feat_hbm, ids_hbm, inv_hbm, out_hbm,
             fbuf, idsb, invb, zbuf,
             acc, feat_sem, scat_sem, flush_sem, zero_sem):
    c = lax.axis_index("c")
    s = lax.axis_index("s")
    row_base = s * TPB
    col_base = c * (F // NCORE)
    stripe = s * RPT

    pltpu.sync_copy(ids_hbm.at[s], idsb)
    pltpu.sync_copy(inv_hbm.at[s], invb)

    zv = jnp.zeros((LANES,), jnp.float32)

    def zrow(r, carry):
        for jj in range(FC // LANES):
            zbuf[r, pl.ds(jj * LANES, LANES)] = zv
        return carry

    lax.fori_loop(0, ZR, zrow, 0)

    def _feat_dma(buf_idx, gq):
        gqw = lax.rem(gq, NCHUNK * NQ)
        fch = lax.div(gqw, NQ)
        q = lax.rem(gqw, NQ)
        return pltpu.make_async_copy(
            feat_hbm.at[pl.ds(row_base + q * QR, QR),
                        pl.ds(col_base + fch * FC, FC)],
            fbuf.at[buf_idx], feat_sem)

    def _scale(buf_idx, q):
        def grp(g, carry):
            invv = invb[pl.ds(q * QR + g * LANES, LANES)]
            for lane in range(LANES):
                sv = jnp.full((LANES,), invv[lane])
                r = g * LANES + lane
                for vj in range(FC // LANES):
                    sl = pl.ds(vj * LANES, LANES)
                    fbuf[buf_idx, r, sl] = fbuf[buf_idx, r, sl] * sv
            return carry

        lax.fori_loop(0, QR // LANES, grp, 0)

    def _fire_scat(buf_idx, q):
        pltpu.async_copy(
            fbuf.at[buf_idx], acc.at[idsb.at[q]], scat_sem, add=True)

    def _drain_scat():
        pltpu.make_async_copy(
            fbuf.at[0], acc.at[idsb.at[0]], scat_sem).wait()

    for z in range(RPT // ZR):
        pltpu.sync_copy(zbuf, acc.at[pl.ds(stripe + z * ZR, ZR)])
    _feat_dma(0, 0).start()
    plsc.subcore_barrier()

    def chunk_body(fchunk, carry):
        gq0 = fchunk * NQ
        col0 = col_base + fchunk * FC

        _feat_dma(0, gq0).wait()
        _scale(0, 0)

        @pl.when(fchunk >= 1)
        def _():
            for h in range(4):
                pltpu.make_async_copy(
                    acc.at[pl.ds(stripe + h * HRPT, HRPT)],
                    out_hbm.at[pl.ds(stripe + h * HRPT, HRPT),
                               pl.ds(col0, FC)],
                    flush_sem).wait()
                for z in range(HRPT // ZR):
                    pltpu.async_copy(
                        zbuf,
                        acc.at[pl.ds(stripe + h * HRPT + z * ZR, ZR)],
                        zero_sem)
            for _ in range(4 * (HRPT // ZR)):
                pltpu.make_async_copy(
                    zbuf, acc.at[pl.ds(stripe, ZR)], zero_sem).wait()

        plsc.subcore_barrier()

        _fire_scat(0, 0)
        _feat_dma(1, gq0 + 1).start()
        for q in range(1, NQ):
            b = q % 2
            _feat_dma(b, gq0 + q).wait()
            _scale(b, q)
            _fire_scat(b, q)
            _drain_scat()
            _feat_dma(1 - b, gq0 + q + 1).start()
        _drain_scat()

        plsc.subcore_barrier()
        for h in range(4):
            pltpu.async_copy(
                acc.at[pl.ds(stripe + h * HRPT, HRPT)],
                out_hbm.at[pl.ds(stripe + h * HRPT, HRPT),
                           pl.ds(col0, FC)],
                flush_sem)
        return carry

    lax.fori_loop(0, NCHUNK, chunk_body, 0)

    for h in range(4):
        pltpu.make_async_copy(
            acc.at[pl.ds(stripe + h * HRPT, HRPT)],
            out_hbm.at[pl.ds(stripe + h * HRPT, HRPT), pl.ds(col_base, FC)],
            flush_sem).wait()
    _feat_dma(0, 0).wait()


_sc_scatter = functools.partial(
    pl.kernel,
    out_type=jax.ShapeDtypeStruct((CPAD, F), jnp.float32),
    mesh=plsc.VectorSubcoreMesh(core_axis_name="c", subcore_axis_name="s"),
    scratch_types=[
        pltpu.VMEM((2, QR, FC), jnp.float32),
        pltpu.VMEM((NSB, SB), jnp.int32),
        pltpu.VMEM((TPB,), jnp.float32),
        pltpu.VMEM((ZR, FC), jnp.float32),
        pltpu.VMEM_SHARED((CPAD, FC), jnp.float32),
        pltpu.SemaphoreType.DMA,
        pltpu.SemaphoreType.DMA,
        pltpu.SemaphoreType.DMA,
        pltpu.SemaphoreType.DMA,
    ],
)(_sc_body)


def kernel(features_v, features_r, vis_memory, ir_memory, ids_v, ids_r):
    inv_v = _norms(features_v).reshape(NSUB_CORES, TPB)
    inv_r = _norms(features_r).reshape(NSUB_CORES, TPB)
    sums_v = _sc_scatter(features_v, ids_v.reshape(NSUB_CORES, NSB, SB), inv_v)
    sums_r = _sc_scatter(features_r, ids_r.reshape(NSUB_CORES, NSB, SB), inv_r)
    vis_new = _finalize(sums_v, vis_memory)
    ir_new = _finalize(sums_r, ir_memory)
    return (vis_new, ir_new)
